# baseline (speedup 1.0000x reference)
import functools

import jax
import jax.numpy as jnp
from jax import lax
from jax.experimental import pallas as pl
from jax.experimental.pallas import tpu as pltpu
from jax.experimental.pallas import tpu_sc as plsc

N = 10000
D = 128
H = 256
K = 16
EPS = 1e-5
NPAD = 10240
NK = N * K
NH = 5000
G = NPAD // 128
_BIG = 3.0e38

KNN_R = 200
EB = 3200
NB = EB // K

SC_NC = 2
SC_NW = 32
SC_CHUNK = 128


def _knn_body(xb_ref, xall_ref, wcat_ref, bcat_ref,
              dist_ref, gidx_ref, gsel_ref, a_ref, b_ref, sq_ref, r0):
    xb = xb_ref[...]
    xall = xall_ref[...]
    ab = (
        jnp.dot(xb, wcat_ref[...], preferred_element_type=jnp.float32)
        + bcat_ref[...]
    )
    a_ref[...] = ab[:, :H]
    b_ref[...] = ab[:, H:]

    @pl.when(pl.program_id(0) == 0)
    def _():
        sq_ref[...] = jnp.sum(xall * xall, axis=1)[None, :]

    mm = lax.dot_general(
        xb, xall, (((1,), (1,)), ((), ())),
        preferred_element_type=jnp.float32,
    )
    sqb = jnp.sum(xb * xb, axis=1)
    col = lax.broadcasted_iota(jnp.int32, (KNN_R, NPAD), 1)
    dist = sqb[:, None] - 2.0 * mm + sq_ref[...]
    dist = jnp.where(col >= N, _BIG, dist)
    dist_ref[...] = dist.reshape(KNN_R * G, 128)
    M = jnp.min(dist.reshape(KNN_R, G, 128), axis=2)
    giota = lax.broadcasted_iota(jnp.int32, (KNN_R, G), 1).astype(jnp.float32)
    riota = lax.broadcasted_iota(jnp.int32, (KNN_R, K), 0)
    gs = []
    for _ in range(K):
        m = jnp.min(M, axis=1, keepdims=True)
        g = jnp.min(jnp.where(M == m, giota, _BIG), axis=1, keepdims=True)
        M = jnp.where(giota == g, _BIG, M)
        gs.append(g)
    gsel = jnp.concatenate(gs, axis=1).astype(jnp.int32)
    gsel_ref[...] = gsel
    base = (pl.program_id(0) * KNN_R + riota) * G
    gidx_ref[...] = base + gsel


def _knn_call(x_pad, wcat, bcat, r0):
    steps = NH // KNN_R
    return pl.pallas_call(
        functools.partial(_knn_body, r0=r0),
        grid=(steps,),
        in_specs=[
            pl.BlockSpec((KNN_R, D), lambda i, r=r0 // KNN_R: (r + i, 0)),
            pl.BlockSpec((NPAD, D), lambda i: (0, 0)),
            pl.BlockSpec((D, 2 * H), lambda i: (0, 0)),
            pl.BlockSpec((1, 2 * H), lambda i: (0, 0)),
        ],
        out_specs=[
            pl.BlockSpec((KNN_R * G, 128), lambda i: (i, 0)),
            pl.BlockSpec((KNN_R, K), lambda i: (i, 0)),
            pl.BlockSpec((KNN_R, K), lambda i: (i, 0)),
            pl.BlockSpec((KNN_R, H), lambda i: (i, 0)),
            pl.BlockSpec((KNN_R, H), lambda i: (i, 0)),
        ],
        out_shape=[
            jax.ShapeDtypeStruct((NH * G, 128), jnp.float32),
            jax.ShapeDtypeStruct((NH, K), jnp.int32),
            jax.ShapeDtypeStruct((NH, K), jnp.int32),
            jax.ShapeDtypeStruct((NH, H), jnp.float32),
            jax.ShapeDtypeStruct((NH, H), jnp.float32),
        ],
        scratch_shapes=[pltpu.VMEM((1, NPAD), jnp.float32)],
    )(x_pad, x_pad, wcat, bcat)


def _sc_gather_body(table_hbm, idx_hbm, out_hbm,
                    idx0, idx1, buf0, buf1, sem0, sem1, nchunk):
    w = lax.axis_index("s") * SC_NC + lax.axis_index("c")
    idxv = (idx0, idx1)
    bufv = (buf0, buf1)
    sems = (sem0, sem1)
    iters = -(-nchunk // SC_NW)

    def start(i):
        c = i * SC_NW + w
        p = i % 2

        @pl.when(c < nchunk)
        def _():
            pltpu.sync_copy(idx_hbm.at[pl.ds(c * SC_CHUNK, SC_CHUNK)], idxv[p])

        return pltpu.async_copy(table_hbm.at[idxv[p]], bufv[p], sems[p])

    handles = [None] * iters
    handles[0] = start(0)
    for i in range(iters):
        if i + 1 < iters:
            handles[i + 1] = start(i + 1)
        c = i * SC_NW + w
        handles[i].wait()

        @pl.when(c < nchunk)
        def _():
            pltpu.sync_copy(bufv[i % 2], out_hbm.at[pl.ds(c * SC_CHUNK, SC_CHUNK)])


def _sc_gather(table, idx_flat, width):
    nk = idx_flat.shape[0]
    mesh = plsc.VectorSubcoreMesh(core_axis_name="c", subcore_axis_name="s")
    f = functools.partial(
        pl.kernel,
        mesh=mesh,
        out_type=jax.ShapeDtypeStruct((nk, width), jnp.float32),
        scratch_types=[
            pltpu.VMEM((SC_CHUNK,), jnp.int32),
            pltpu.VMEM((SC_CHUNK,), jnp.int32),
            pltpu.VMEM((SC_CHUNK, width), jnp.float32),
            pltpu.VMEM((SC_CHUNK, width), jnp.float32),
            pltpu.SemaphoreType.DMA,
            pltpu.SemaphoreType.DMA,
        ],
    )(functools.partial(_sc_gather_body, nchunk=nk // SC_CHUNK))
    return f(table, idx_flat)


def _topk_body(cand_ref, idx_ref):
    c3 = cand_ref[...]
    jiota = lax.broadcasted_iota(jnp.int32, (K, KNN_R, 128), 0)
    lane = lax.broadcasted_iota(jnp.int32, (K, KNN_R, 128), 2)
    col3 = (jiota * 128 + lane).astype(jnp.float32)
    cols = []
    for _ in range(K):
        mj = jnp.min(c3, axis=0, keepdims=True)
        m = jnp.min(mj, axis=2, keepdims=True)
        cc = jnp.where(c3 == m, col3, _BIG)
        cj = jnp.min(cc, axis=0, keepdims=True)
        ci = jnp.min(cj, axis=2, keepdims=True)
        c3 = jnp.where(col3 == ci, _BIG, c3)
        cols.append(ci)
    idxs = jnp.concatenate(cols, axis=2)
    idx_ref[...] = idxs.reshape(KNN_R, K).astype(jnp.int32)


def _topk_call(cand):
    steps = NH // KNN_R
    return pl.pallas_call(
        _topk_body,
        grid=(steps,),
        in_specs=[
            pl.BlockSpec((K, KNN_R, 128), lambda i: (0, i, 0)),
        ],
        out_specs=pl.BlockSpec((KNN_R, K), lambda i: (i, 0)),
        out_shape=jax.ShapeDtypeStruct((NH, K), jnp.int32),
    )(cand)


def _stats_rows(z):
    s = jnp.sum(z, axis=0)
    q = jnp.sum(z * z, axis=0)
    return jnp.concatenate(
        [s[None, :], q[None, :], jnp.zeros((6, H), jnp.float32)], axis=0
    )


def _acc_stats(st_ref, st):
    @pl.when(pl.program_id(0) == 0)
    def _():
        st_ref[...] = st

    @pl.when(pl.program_id(0) != 0)
    def _():
        st_ref[...] = st_ref[...] + st


def _s1_body(zr_ref, a_ref, st_ref):
    z = (
        zr_ref[...].reshape(NB, K, H) + a_ref[...].reshape(NB, 1, H)
    ).reshape(EB, H)
    _acc_stats(st_ref, _stats_rows(z))


def _s1_call(z1raw, a):
    steps = z1raw.shape[0] // EB
    return pl.pallas_call(
        _s1_body,
        grid=(steps,),
        in_specs=[
            pl.BlockSpec((EB, H), lambda i: (i, 0)),
            pl.BlockSpec((NB, H), lambda i: (i, 0)),
        ],
        out_specs=pl.BlockSpec((8, H), lambda i: (0, 0)),
        out_shape=jax.ShapeDtypeStruct((8, H), jnp.float32),
    )(z1raw, a)


def _mlp1_body(zr_ref, a_ref, sc_ref, sh_ref, w_ref, b_ref, z2_ref, st_ref):
    z1 = (
        zr_ref[...].reshape(NB, K, H) + a_ref[...].reshape(NB, 1, H)
    ).reshape(EB, H)
    h = jnp.maximum(z1 * sc_ref[...] + sh_ref[...], 0.0)
    z2 = jnp.dot(h.astype(jnp.bfloat16), w_ref[...],
                 preferred_element_type=jnp.float32) + b_ref[...]
    z2_ref[...] = z2.astype(jnp.bfloat16)
    _acc_stats(st_ref, _stats_rows(z2))


def _mlp2_body(z_ref, sc_ref, sh_ref, w_ref, b_ref, z3_ref, st_ref):
    h = jnp.maximum(
        z_ref[...].astype(jnp.float32) * sc_ref[...] + sh_ref[...], 0.0)
    z3 = jnp.dot(h.astype(jnp.bfloat16), w_ref[...],
                 preferred_element_type=jnp.float32) + b_ref[...]
    z3_ref[...] = z3.astype(jnp.bfloat16)
    _acc_stats(st_ref, _stats_rows(z3))


def _mlp1_call(z1raw, a, scale, shift, w, b):
    steps = z1raw.shape[0] // EB
    return pl.pallas_call(
        _mlp1_body,
        grid=(steps,),
        in_specs=[
            pl.BlockSpec((EB, H), lambda i: (i, 0)),
            pl.BlockSpec((NB, H), lambda i: (i, 0)),
            pl.BlockSpec((1, H), lambda i: (0, 0)),
            pl.BlockSpec((1, H), lambda i: (0, 0)),
            pl.BlockSpec((H, H), lambda i: (0, 0)),
            pl.BlockSpec((1, H), lambda i: (0, 0)),
        ],
        out_specs=[
            pl.BlockSpec((EB, H), lambda i: (i, 0)),
            pl.BlockSpec((8, H), lambda i: (0, 0)),
        ],
        out_shape=[
            jax.ShapeDtypeStruct((z1raw.shape[0], H), jnp.bfloat16),
            jax.ShapeDtypeStruct((8, H), jnp.float32),
        ],
    )(z1raw, a, scale, shift, w, b)


def _mlp2_call(z, scale, shift, w, b):
    steps = z.shape[0] // EB
    return pl.pallas_call(
        _mlp2_body,
        grid=(steps,),
        in_specs=[
            pl.BlockSpec((EB, H), lambda i: (i, 0)),
            pl.BlockSpec((1, H), lambda i: (0, 0)),
            pl.BlockSpec((1, H), lambda i: (0, 0)),
            pl.BlockSpec((H, H), lambda i: (0, 0)),
            pl.BlockSpec((1, H), lambda i: (0, 0)),
        ],
        out_specs=[
            pl.BlockSpec((EB, H), lambda i: (i, 0)),
            pl.BlockSpec((8, H), lambda i: (0, 0)),
        ],
        out_shape=[
            jax.ShapeDtypeStruct((z.shape[0], H), jnp.bfloat16),
            jax.ShapeDtypeStruct((8, H), jnp.float32),
        ],
    )(z, scale, shift, w, b)


def _out_body(z_ref, sc_ref, sh_ref, o_ref):
    h = jnp.maximum(
        z_ref[...].astype(jnp.float32) * sc_ref[...] + sh_ref[...], 0.0)
    o_ref[...] = jnp.sum(h.reshape(NB, K, H), axis=1)


def _out_call(z3, scale, shift):
    steps = z3.shape[0] // EB
    return pl.pallas_call(
        _out_body,
        grid=(steps,),
        in_specs=[
            pl.BlockSpec((EB, H), lambda i: (i, 0)),
            pl.BlockSpec((1, H), lambda i: (0, 0)),
            pl.BlockSpec((1, H), lambda i: (0, 0)),
        ],
        out_specs=pl.BlockSpec((NB, H), lambda i: (i, 0)),
        out_shape=jax.ShapeDtypeStruct((z3.shape[0] // K, H), jnp.float32),
    )(z3, scale, shift)


def _bn_coeffs(st, gamma, beta):
    mu = st[0] / NK
    var = st[1] / NK - mu * mu
    scale = gamma / jnp.sqrt(var + EPS)
    shift = beta - mu * scale
    return scale[None, :], shift[None, :]


def _half_knn(x_pad, wcat, bcat, r0):
    dist, gidx, gsel, a, btab = _knn_call(x_pad, wcat, bcat, r0)
    gidx_t = gidx.T.reshape(NH * K)
    cand = _sc_gather(dist, gidx_t, 128).reshape(K, NH, 128)
    loc = _topk_call(cand)
    idx = jnp.take_along_axis(gsel, loc >> 7, axis=1) * 128 + (loc & 127)
    return idx, a, btab


def kernel(X, W1, b1, g1, be1, W2, b2, g2, be2, W3, b3, g3, be3):
    x_pad = jnp.pad(X, ((0, NPAD - N), (0, 0)))
    w1a, w1b = W1[:D], W1[D:]
    wcat = jnp.concatenate([w1a - w1b, w1b], axis=1)
    bcat = jnp.concatenate([b1, jnp.zeros_like(b1)])[None, :]

    idx0, a0, b0 = _half_knn(x_pad, wcat, bcat, 0)
    idx1, a1, b1t = _half_knn(x_pad, wcat, bcat, NH)
    btab = jnp.concatenate([b0, b1t], axis=0)

    z1raw0 = _sc_gather(btab, idx0.reshape(NH * K), H)
    z1raw1 = _sc_gather(btab, idx1.reshape(NH * K), H)

    st1 = _s1_call(z1raw0, a0) + _s1_call(z1raw1, a1)
    sc1, sh1 = _bn_coeffs(st1, g1, be1)
    w2b = W2.astype(jnp.bfloat16)
    z2_0, st2a = _mlp1_call(z1raw0, a0, sc1, sh1, w2b, b2[None, :])
    z2_1, st2b = _mlp1_call(z1raw1, a1, sc1, sh1, w2b, b2[None, :])
    sc2, sh2 = _bn_coeffs(st2a + st2b, g2, be2)
    w3b = W3.astype(jnp.bfloat16)
    z3_0, st3a = _mlp2_call(z2_0, sc2, sh2, w3b, b3[None, :])
    z3_1, st3b = _mlp2_call(z2_1, sc2, sh2, w3b, b3[None, :])
    sc3, sh3 = _bn_coeffs(st3a + st3b, g3, be3)
    o0 = _out_call(z3_0, sc3, sh3)
    o1 = _out_call(z3_1, sc3, sh3)
    return jnp.concatenate([o0, o1], axis=0)


# speedup vs baseline: 1.1029x; 1.1029x over previous
import functools

import jax
import jax.numpy as jnp
from jax import lax
from jax.experimental import pallas as pl
from jax.experimental.pallas import tpu as pltpu
from jax.experimental.pallas import tpu_sc as plsc

N = 10000
D = 128
H = 256
K = 16
EPS = 1e-5
NPAD = 10240
NK = N * K

KNN_R = 400
KNN_STEPS = N // KNN_R

EB = 6400
NB = EB // K
EB_STEPS = NK // EB

SC_NC = 2
SC_NS = 16
SC_NW = SC_NC * SC_NS
SC_CHUNK = 128
SC_NCHUNK = NK // SC_CHUNK
SC_ITERS = -(-SC_NCHUNK // SC_NW)


G = NPAD // 128
_BIG = 3.0e38


def _knn_body(xb_ref, xall_ref, wcat_ref, bcat_ref, dist_ref, gidx_ref, gsel_ref, a_ref, b_ref, sq_ref):
    xb = xb_ref[...]
    xall = xall_ref[...]
    ab = (
        jnp.dot(xb, wcat_ref[...], preferred_element_type=jnp.float32)
        + bcat_ref[...]
    )
    a_ref[...] = ab[:, :H]
    b_ref[...] = ab[:, H:]

    @pl.when(pl.program_id(0) == 0)
    def _():
        ci = lax.broadcasted_iota(jnp.int32, (1, NPAD), 1)
        sq = jnp.sum(xall * xall, axis=1)[None, :]
        sq_ref[...] = jnp.where(ci >= N, _BIG, sq)

    mm = lax.dot_general(
        xb, xall, (((1,), (1,)), ((), ())),
        preferred_element_type=jnp.float32,
    )
    sqb = jnp.sum(xb * xb, axis=1)
    dist = sqb[:, None] - 2.0 * mm + sq_ref[...]
    dist_ref[...] = dist.reshape(KNN_R * G, 128)
    M = jnp.min(dist.reshape(KNN_R, G, 128), axis=2)
    giota = lax.broadcasted_iota(jnp.int32, (KNN_R, G), 1).astype(jnp.float32)
    riota = lax.broadcasted_iota(jnp.int32, (KNN_R, K), 0)
    gs = []
    for _ in range(K):
        m = jnp.min(M, axis=1, keepdims=True)
        g = jnp.min(jnp.where(M == m, giota, _BIG), axis=1, keepdims=True)
        M = jnp.where(giota == g, _BIG, M)
        gs.append(g)
    gsel = jnp.concatenate(gs, axis=1).astype(jnp.int32)
    gsel_ref[...] = gsel
    base = (pl.program_id(0) * KNN_R + riota) * G
    gidx_ref[...] = base + gsel


def _knn_call(x_pad, wcat, bcat):
    return pl.pallas_call(
        _knn_body,
        grid=(KNN_STEPS,),
        in_specs=[
            pl.BlockSpec((KNN_R, D), lambda i: (i, 0)),
            pl.BlockSpec((NPAD, D), lambda i: (0, 0)),
            pl.BlockSpec((D, 2 * H), lambda i: (0, 0)),
            pl.BlockSpec((1, 2 * H), lambda i: (0, 0)),
        ],
        out_specs=[
            pl.BlockSpec((KNN_R * G, 128), lambda i: (i, 0)),
            pl.BlockSpec((KNN_R, K), lambda i: (i, 0)),
            pl.BlockSpec((KNN_R, K), lambda i: (i, 0)),
            pl.BlockSpec((KNN_R, H), lambda i: (i, 0)),
            pl.BlockSpec((KNN_R, H), lambda i: (i, 0)),
        ],
        out_shape=[
            jax.ShapeDtypeStruct((N * G, 128), jnp.float32),
            jax.ShapeDtypeStruct((N, K), jnp.int32),
            jax.ShapeDtypeStruct((N, K), jnp.int32),
            jax.ShapeDtypeStruct((N, H), jnp.float32),
            jax.ShapeDtypeStruct((N, H), jnp.float32),
        ],
        scratch_shapes=[pltpu.VMEM((1, NPAD), jnp.float32)],
    )(x_pad, x_pad, wcat, bcat)


def _topk_body(cand_ref, idx_ref):
    c3 = cand_ref[...]
    jiota = lax.broadcasted_iota(jnp.int32, (K, KNN_R, 128), 0)
    lane = lax.broadcasted_iota(jnp.int32, (K, KNN_R, 128), 2)
    col3 = (jiota * 128 + lane).astype(jnp.float32)
    cols = []
    for _ in range(K):
        mj = jnp.min(c3, axis=0, keepdims=True)
        m = jnp.min(mj, axis=2, keepdims=True)
        cc = jnp.where(c3 == m, col3, _BIG)
        cj = jnp.min(cc, axis=0, keepdims=True)
        ci = jnp.min(cj, axis=2, keepdims=True)
        c3 = jnp.where(col3 == ci, _BIG, c3)
        cols.append(ci)
    idxs = jnp.concatenate(cols, axis=2)
    idx_ref[...] = idxs.reshape(KNN_R, K).astype(jnp.int32)


def _topk_call(cand):
    return pl.pallas_call(
        _topk_body,
        grid=(KNN_STEPS,),
        in_specs=[
            pl.BlockSpec((K, KNN_R, 128), lambda i: (0, i, 0)),
        ],
        out_specs=pl.BlockSpec((KNN_R, K), lambda i: (i, 0)),
        out_shape=jax.ShapeDtypeStruct((N, K), jnp.int32),
    )(cand)


def _sc_gather_body(table_hbm, idx_hbm, out_hbm,
                    idx0, idx1, buf0, buf1, sem0, sem1):
    w = lax.axis_index("s") * SC_NC + lax.axis_index("c")
    idxv = (idx0, idx1)
    bufv = (buf0, buf1)
    sems = (sem0, sem1)

    def start(i):
        c = i * SC_NW + w
        p = i % 2

        @pl.when(c < SC_NCHUNK)
        def _():
            pltpu.sync_copy(idx_hbm.at[pl.ds(c * SC_CHUNK, SC_CHUNK)], idxv[p])

        return pltpu.async_copy(table_hbm.at[idxv[p]], bufv[p], sems[p])

    handles = [None] * SC_ITERS
    handles[0] = start(0)
    for i in range(SC_ITERS):
        if i + 1 < SC_ITERS:
            handles[i + 1] = start(i + 1)
        c = i * SC_NW + w
        handles[i].wait()

        @pl.when(c < SC_NCHUNK)
        def _():
            pltpu.sync_copy(bufv[i % 2], out_hbm.at[pl.ds(c * SC_CHUNK, SC_CHUNK)])


def _sc_gather(table, idx_flat, width):
    mesh = plsc.VectorSubcoreMesh(core_axis_name="c", subcore_axis_name="s")
    f = functools.partial(
        pl.kernel,
        mesh=mesh,
        out_type=jax.ShapeDtypeStruct((NK, width), jnp.float32),
        scratch_types=[
            pltpu.VMEM((SC_CHUNK,), jnp.int32),
            pltpu.VMEM((SC_CHUNK,), jnp.int32),
            pltpu.VMEM((SC_CHUNK, width), jnp.float32),
            pltpu.VMEM((SC_CHUNK, width), jnp.float32),
            pltpu.SemaphoreType.DMA,
            pltpu.SemaphoreType.DMA,
        ],
    )(_sc_gather_body)
    return f(table, idx_flat)


def _stats_rows(z):
    s = jnp.sum(z, axis=0)
    q = jnp.sum(z * z, axis=0)
    return jnp.concatenate(
        [s[None, :], q[None, :], jnp.zeros((6, H), jnp.float32)], axis=0
    )


def _acc_stats(st_ref, st):
    @pl.when(pl.program_id(0) == 0)
    def _():
        st_ref[...] = st

    @pl.when(pl.program_id(0) != 0)
    def _():
        st_ref[...] = st_ref[...] + st


def _s1_body(zr_ref, a_ref, st_ref):
    z = (
        zr_ref[...].reshape(NB, K, H) + a_ref[...].reshape(NB, 1, H)
    ).reshape(EB, H)
    _acc_stats(st_ref, _stats_rows(z))


def _s1_call(z1raw, a):
    return pl.pallas_call(
        _s1_body,
        grid=(EB_STEPS,),
        in_specs=[
            pl.BlockSpec((EB, H), lambda i: (i, 0)),
            pl.BlockSpec((NB, H), lambda i: (i, 0)),
        ],
        out_specs=pl.BlockSpec((8, H), lambda i: (0, 0)),
        out_shape=jax.ShapeDtypeStruct((8, H), jnp.float32),
    )(z1raw, a)


def _mlp1_body(zr_ref, a_ref, sc_ref, sh_ref, w_ref, b_ref, z2_ref, st_ref):
    z1 = (
        zr_ref[...].reshape(NB, K, H) + a_ref[...].reshape(NB, 1, H)
    ).reshape(EB, H)
    h = jnp.maximum(z1 * sc_ref[...] + sh_ref[...], 0.0)
    z2 = jnp.dot(h.astype(jnp.bfloat16), w_ref[...],
                 preferred_element_type=jnp.float32) + b_ref[...]
    z2_ref[...] = z2.astype(jnp.bfloat16)
    _acc_stats(st_ref, _stats_rows(z2))


def _mlp2_body(z_ref, sc_ref, sh_ref, w_ref, b_ref, z3_ref, st_ref):
    h = jnp.maximum(
        z_ref[...].astype(jnp.float32) * sc_ref[...] + sh_ref[...], 0.0)
    z3 = jnp.dot(h.astype(jnp.bfloat16), w_ref[...],
                 preferred_element_type=jnp.float32) + b_ref[...]
    z3_ref[...] = z3.astype(jnp.bfloat16)
    _acc_stats(st_ref, _stats_rows(z3))


def _mlp1_call(z1raw, a, scale, shift, w, b):
    return pl.pallas_call(
        _mlp1_body,
        grid=(EB_STEPS,),
        in_specs=[
            pl.BlockSpec((EB, H), lambda i: (i, 0)),
            pl.BlockSpec((NB, H), lambda i: (i, 0)),
            pl.BlockSpec((1, H), lambda i: (0, 0)),
            pl.BlockSpec((1, H), lambda i: (0, 0)),
            pl.BlockSpec((H, H), lambda i: (0, 0)),
            pl.BlockSpec((1, H), lambda i: (0, 0)),
        ],
        out_specs=[
            pl.BlockSpec((EB, H), lambda i: (i, 0)),
            pl.BlockSpec((8, H), lambda i: (0, 0)),
        ],
        out_shape=[
            jax.ShapeDtypeStruct((NK, H), jnp.bfloat16),
            jax.ShapeDtypeStruct((8, H), jnp.float32),
        ],
    )(z1raw, a, scale, shift, w, b)


def _mlp2_call(z, scale, shift, w, b):
    return pl.pallas_call(
        _mlp2_body,
        grid=(EB_STEPS,),
        in_specs=[
            pl.BlockSpec((EB, H), lambda i: (i, 0)),
            pl.BlockSpec((1, H), lambda i: (0, 0)),
            pl.BlockSpec((1, H), lambda i: (0, 0)),
            pl.BlockSpec((H, H), lambda i: (0, 0)),
            pl.BlockSpec((1, H), lambda i: (0, 0)),
        ],
        out_specs=[
            pl.BlockSpec((EB, H), lambda i: (i, 0)),
            pl.BlockSpec((8, H), lambda i: (0, 0)),
        ],
        out_shape=[
            jax.ShapeDtypeStruct((NK, H), jnp.bfloat16),
            jax.ShapeDtypeStruct((8, H), jnp.float32),
        ],
    )(z, scale, shift, w, b)


def _out_body(z_ref, sc_ref, sh_ref, o_ref):
    h = jnp.maximum(
        z_ref[...].astype(jnp.float32) * sc_ref[...] + sh_ref[...], 0.0)
    o_ref[...] = jnp.sum(h.reshape(NB, K, H), axis=1)


def _out_call(z3, scale, shift):
    return pl.pallas_call(
        _out_body,
        grid=(EB_STEPS,),
        in_specs=[
            pl.BlockSpec((EB, H), lambda i: (i, 0)),
            pl.BlockSpec((1, H), lambda i: (0, 0)),
            pl.BlockSpec((1, H), lambda i: (0, 0)),
        ],
        out_specs=pl.BlockSpec((NB, H), lambda i: (i, 0)),
        out_shape=jax.ShapeDtypeStruct((N, H), jnp.float32),
    )(z3, scale, shift)


def _bn_coeffs(st, gamma, beta):
    mu = st[0] / NK
    var = st[1] / NK - mu * mu
    scale = gamma / jnp.sqrt(var + EPS)
    shift = beta - mu * scale
    return scale[None, :], shift[None, :]


def kernel(X, W1, b1, g1, be1, W2, b2, g2, be2, W3, b3, g3, be3):
    x_pad = jnp.pad(X, ((0, NPAD - N), (0, 0)))
    w1a, w1b = W1[:D], W1[D:]
    wcat = jnp.concatenate([w1a - w1b, w1b], axis=1)
    bcat = jnp.concatenate([b1, jnp.zeros_like(b1)])[None, :]

    dist, gidx, gsel, a, btab = _knn_call(x_pad, wcat, bcat)
    gidx_t = gidx.T.reshape(NK)
    cand = _sc_gather(dist, gidx_t, 128).reshape(K, N, 128)
    loc = _topk_call(cand)
    idx = jnp.take_along_axis(gsel, loc >> 7, axis=1) * 128 + (loc & 127)
    idx_flat = idx.reshape(NK)

    z1raw = _sc_gather(btab, idx_flat, H)

    st1 = _s1_call(z1raw, a)
    sc1, sh1 = _bn_coeffs(st1, g1, be1)
    z2, st2 = _mlp1_call(z1raw, a, sc1, sh1,
                         W2.astype(jnp.bfloat16), b2[None, :])
    sc2, sh2 = _bn_coeffs(st2, g2, be2)
    z3, st3 = _mlp2_call(z2, sc2, sh2,
                         W3.astype(jnp.bfloat16), b3[None, :])
    sc3, sh3 = _bn_coeffs(st3, g3, be3)
    return _out_call(z3, sc3, sh3)
